# baseline (speedup 1.0000x reference)
import functools

import jax
import jax.numpy as jnp
from jax import lax
from jax.experimental import pallas as pl
from jax.experimental.pallas import tpu as pltpu
from jax.experimental.pallas import tpu_sc as plsc

_NUM_EMB = 16909
_EMB_DIM = 200
_OUT_DIM = 512
_BATCH = 4
_SEQ = 16906


_BM = 512


def _proj_body(emb_ref, w_ref, b_ref, out_ref):
    out_ref[...] = (
        jnp.dot(emb_ref[...], w_ref[...], preferred_element_type=jnp.float32)
        + b_ref[...]
    )


def _project(emb, w, b):
    return pl.pallas_call(
        _proj_body,
        grid=(pl.cdiv(_NUM_EMB, _BM),),
        in_specs=[
            pl.BlockSpec((_BM, _EMB_DIM), lambda i: (i, 0)),
            pl.BlockSpec((_EMB_DIM, _OUT_DIM), lambda i: (0, 0)),
            pl.BlockSpec((1, _OUT_DIM), lambda i: (0, 0)),
        ],
        out_specs=pl.BlockSpec((_BM, _OUT_DIM), lambda i: (i, 0)),
        out_shape=jax.ShapeDtypeStruct((_NUM_EMB, _OUT_DIM), jnp.float32),
    )(emb, w, b.reshape(1, _OUT_DIM))


_NW = 32
_WROWS = 112
_SEQ_PAD = 16912
_WIN_PB = 151
_TAIL_W = 150
_TAIL_START = _TAIL_W * _WROWS
_TAIL_ROWS = _SEQ - _TAIL_START
_MAXWIN = 19

_mesh = plsc.VectorSubcoreMesh(core_axis_name="c", subcore_axis_name="s")


@functools.partial(
    pl.kernel,
    out_type=jax.ShapeDtypeStruct((_BATCH, _SEQ, _OUT_DIM), jnp.float32),
    mesh=_mesh,
    scratch_types=[
        pltpu.VMEM((_WROWS,), jnp.int32),
        pltpu.VMEM((_WROWS, _OUT_DIM), jnp.float32),
        pltpu.SemaphoreType.DMA,
    ],
)
def _gather(table_hbm, idx_hbm, out_hbm, idx_v, rows_v, sem):
    wid = lax.axis_index("s") * 2 + lax.axis_index("c")
    bi = wid // 8
    j = wid % 8
    nwin = jnp.where(j < _WIN_PB % 8, _MAXWIN, _MAXWIN - 1)

    @pl.loop(0, _MAXWIN)
    def _win(i):
        @pl.when(i < nwin)
        def _():
            w = j + 8 * i
            start = w * _WROWS
            pltpu.sync_copy(
                idx_hbm.at[pl.ds(bi * _SEQ_PAD + start, _WROWS)], idx_v)

            @pl.loop(0, _WROWS, step=16)
            def _norm(jj):
                v = idx_v[pl.ds(jj, 16)]
                idx_v[pl.ds(jj, 16)] = lax.rem(v + _NUM_EMB, _NUM_EMB)

            pltpu.async_copy(table_hbm.at[idx_v], rows_v, sem).wait()

            pltpu.sync_copy(rows_v, out_hbm.at[bi, pl.ds(start, _WROWS)])


def kernel(x, emb, W, b):
    proj = _project(emb, W, b)
    idx = jnp.pad(x, ((0, 0), (0, _SEQ_PAD - _SEQ))).reshape(_BATCH * _SEQ_PAD)
    return _gather(proj, idx)


# speedup vs baseline: 1.9123x; 1.9123x over previous
import functools

import jax
import jax.numpy as jnp
from jax import lax
from jax.experimental import pallas as pl
from jax.experimental.pallas import tpu as pltpu
from jax.experimental.pallas import tpu_sc as plsc

_NUM_EMB = 16909
_EMB_DIM = 200
_OUT_DIM = 512
_BATCH = 4
_SEQ = 16906


_BM = 512


def _proj_body(emb_ref, w_ref, b_ref, out_ref):
    acc = (
        jnp.dot(emb_ref[...], w_ref[...], preferred_element_type=jnp.float32)
        + b_ref[...]
    )
    out_ref[...] = acc.reshape(_BM, 4, 128)


def _project(emb, w, b):
    return pl.pallas_call(
        _proj_body,
        grid=(pl.cdiv(_NUM_EMB, _BM),),
        in_specs=[
            pl.BlockSpec((_BM, _EMB_DIM), lambda i: (i, 0)),
            pl.BlockSpec((_EMB_DIM, _OUT_DIM), lambda i: (0, 0)),
            pl.BlockSpec((1, _OUT_DIM), lambda i: (0, 0)),
        ],
        out_specs=pl.BlockSpec((_BM, 4, 128), lambda i: (i, 0, 0)),
        out_shape=jax.ShapeDtypeStruct((_NUM_EMB, 4, 128), jnp.float32),
    )(emb, w, b.reshape(1, _OUT_DIM))


_NW = 32
_TWIN = 48
_CHUNK = 528
_NWIN = 11
_SEQ_PAD = 16912
_TAIL_T0 = _NW * _CHUNK
_TAIL_N = _SEQ - _TAIL_T0

_mesh = plsc.VectorSubcoreMesh(core_axis_name="c", subcore_axis_name="s")


@functools.partial(
    pl.kernel,
    out_type=jax.ShapeDtypeStruct((_SEQ, 4, _BATCH, 128), jnp.float32),
    mesh=_mesh,
    scratch_types=[
        pltpu.VMEM((_BATCH, _TWIN), jnp.int32),
        pltpu.VMEM((_TWIN, 4, _BATCH, 128), jnp.float32),
        pltpu.SemaphoreType.DMA,
    ],
)
def _gather(table_hbm, idx_hbm, out_hbm, idx_v, slab, sem):
    wid = lax.axis_index("s") * 2 + lax.axis_index("c")
    base = wid * _CHUNK

    def load_norm(b, t0, n):
        pltpu.sync_copy(idx_hbm.at[pl.ds(b * _SEQ_PAD + t0, n)],
                        idx_v.at[b, pl.ds(0, n)])
        for j in range(0, n, 16):
            v = idx_v[b, pl.ds(j, 16)]
            idx_v[b, pl.ds(j, 16)] = lax.rem(v + _NUM_EMB, _NUM_EMB)

    @pl.loop(0, _NWIN)
    def _win(i):
        t0 = base + i * _TWIN
        for b in range(_BATCH):
            load_norm(b, t0, _TWIN)
        copies = [
            pltpu.async_copy(table_hbm.at[idx_v.at[b]],
                             slab.at[:, :, b, :], sem)
            for b in range(_BATCH)
        ]
        for c in copies:
            c.wait()
        pltpu.sync_copy(slab, out_hbm.at[pl.ds(t0, _TWIN)])

    @pl.when(wid == _NW - 1)
    def _tail():
        for b in range(_BATCH):
            load_norm(b, _TAIL_T0, 16)
        copies = [
            pltpu.async_copy(table_hbm.at[idx_v.at[b, pl.ds(0, 16)]],
                             slab.at[pl.ds(0, 16), :, b, :], sem)
            for b in range(_BATCH)
        ]
        for c in copies:
            c.wait()
        pltpu.sync_copy(slab.at[pl.ds(0, _TAIL_N)],
                        out_hbm.at[pl.ds(_TAIL_T0, _TAIL_N)])


def kernel(x, emb, W, b):
    proj = _project(emb, W, b)
    idx = jnp.pad(x, ((0, 0), (0, _SEQ_PAD - _SEQ))).reshape(_BATCH * _SEQ_PAD)
    slab = _gather(proj, idx)
    return slab.transpose(2, 0, 1, 3).reshape(_BATCH, _SEQ, _OUT_DIM)


# speedup vs baseline: 1.9801x; 1.0355x over previous
import functools

import jax
import jax.numpy as jnp
from jax import lax
from jax.experimental import pallas as pl
from jax.experimental.pallas import tpu as pltpu
from jax.experimental.pallas import tpu_sc as plsc

_NUM_EMB = 16909
_EMB_DIM = 200
_OUT_DIM = 512
_BATCH = 4
_SEQ = 16906


_BM = 512


def _proj_body(et_ref, w_ref, b_ref, out_ref):
    acc = lax.dot_general(
        et_ref[...], w_ref[...], (((0,), (0,)), ((), ())),
        preferred_element_type=jnp.float32,
    ) + b_ref[...]
    out_ref[...] = acc.reshape(_BM, 4, 128)


def _project(emb_t, w, b):
    return pl.pallas_call(
        _proj_body,
        grid=(pl.cdiv(_NUM_EMB, _BM),),
        in_specs=[
            pl.BlockSpec((_EMB_DIM, _BM), lambda i: (0, i)),
            pl.BlockSpec((_EMB_DIM, _OUT_DIM), lambda i: (0, 0)),
            pl.BlockSpec((1, _OUT_DIM), lambda i: (0, 0)),
        ],
        out_specs=pl.BlockSpec((_BM, 4, 128), lambda i: (i, 0, 0)),
        out_shape=jax.ShapeDtypeStruct((_NUM_EMB, 4, 128), jnp.float32),
    )(emb_t, w, b.reshape(1, _OUT_DIM))


_NW = 32
_TWIN = 24
_NWIN = 22
_CHUNK = _TWIN * _NWIN
_SEQ_PAD = 16912
_TAIL_T0 = _NW * _CHUNK
_TAIL_N = _SEQ - _TAIL_T0

_mesh = plsc.VectorSubcoreMesh(core_axis_name="c", subcore_axis_name="s")


@functools.partial(
    pl.kernel,
    out_type=jax.ShapeDtypeStruct((_SEQ, 4, _BATCH, 128), jnp.float32),
    mesh=_mesh,
    scratch_types=[
        pltpu.VMEM((2, _BATCH, 32), jnp.int32),
        pltpu.VMEM((2, _TWIN, 4, _BATCH, 128), jnp.float32),
        pltpu.SemaphoreType.DMA,
        pltpu.SemaphoreType.DMA,
        pltpu.SemaphoreType.DMA,
        pltpu.SemaphoreType.DMA,
    ],
)
def _gather(table_hbm, idx_hbm, out_hbm, idx2, slab2, g0, g1, w0, w1):
    gsem = (g0, g1)
    wsem = (w0, w1)
    wid = lax.axis_index("s") * 2 + lax.axis_index("c")
    base = wid * _CHUNK

    def load_norm(i, s, n):
        t0 = base + i * _TWIN
        for b in range(_BATCH):
            pltpu.sync_copy(idx_hbm.at[pl.ds(b * _SEQ_PAD + t0, n)],
                            idx2.at[s, b, pl.ds(0, n)])
        for b in range(_BATCH):
            for j in range(0, 32, 16):
                v = idx2[s, b, pl.ds(j, 16)]
                idx2[s, b, pl.ds(j, 16)] = lax.rem(v + _NUM_EMB, _NUM_EMB)

    def fire_gathers(s):
        for b in range(_BATCH):
            pltpu.async_copy(table_hbm.at[idx2.at[s, b, pl.ds(0, _TWIN)]],
                             slab2.at[s, :, :, b, :], gsem[s])

    def wait_gathers(s):
        for b in range(_BATCH):
            pltpu.make_async_copy(
                table_hbm.at[idx2.at[s, b, pl.ds(0, _TWIN)]],
                slab2.at[s, :, :, b, :], gsem[s]).wait()

    def fire_wb(i, s):
        pltpu.async_copy(slab2.at[s],
                         out_hbm.at[pl.ds(base + i * _TWIN, _TWIN)], wsem[s])

    def wait_wb(s):
        pltpu.make_async_copy(slab2.at[s], out_hbm.at[pl.ds(0, _TWIN)],
                              wsem[s]).wait()

    @pl.loop(0, _NWIN // 2)
    def _pair(k):
        a = 2 * k

        @pl.when(k > 0)
        def _():
            wait_wb(0)
        load_norm(a, 0, _TWIN)
        fire_gathers(0)

        @pl.when(k > 0)
        def _():
            wait_gathers(1)
            fire_wb(a - 1, 1)
            wait_wb(1)
        load_norm(a + 1, 1, _TWIN)
        fire_gathers(1)

        wait_gathers(0)
        fire_wb(a, 0)

    wait_gathers(1)
    fire_wb(_NWIN - 1, 1)
    wait_wb(0)
    wait_wb(1)

    @pl.when(wid == _NW - 1)
    def _tail():
        for b in range(_BATCH):
            pltpu.sync_copy(idx_hbm.at[pl.ds(b * _SEQ_PAD + _TAIL_T0, 16)],
                            idx2.at[0, b, pl.ds(0, 16)])
            v = idx2[0, b, pl.ds(0, 16)]
            idx2[0, b, pl.ds(0, 16)] = lax.rem(v + _NUM_EMB, _NUM_EMB)
        for b in range(_BATCH):
            pltpu.async_copy(table_hbm.at[idx2.at[0, b, pl.ds(0, 16)]],
                             slab2.at[0, pl.ds(0, 16), :, b, :], g0)
        for b in range(_BATCH):
            pltpu.make_async_copy(table_hbm.at[idx2.at[0, b, pl.ds(0, 16)]],
                                  slab2.at[0, pl.ds(0, 16), :, b, :],
                                  g0).wait()
        pltpu.sync_copy(slab2.at[0, pl.ds(0, _TAIL_N)],
                        out_hbm.at[pl.ds(_TAIL_T0, _TAIL_N)])


def kernel(x, emb, W, b):
    proj = _project(emb.T, W, b)
    idx = jnp.pad(x, ((0, 0), (0, _SEQ_PAD - _SEQ))).reshape(_BATCH * _SEQ_PAD)
    slab = _gather(proj, idx)
    return slab.transpose(2, 0, 1, 3).reshape(_BATCH, _SEQ, _OUT_DIM)


# speedup vs baseline: 2.4151x; 1.2196x over previous
import functools

import jax
import jax.numpy as jnp
from jax import lax
from jax.experimental import pallas as pl
from jax.experimental.pallas import tpu as pltpu
from jax.experimental.pallas import tpu_sc as plsc

_NUM_EMB = 16909
_EMB_DIM = 200
_OUT_DIM = 512
_BATCH = 4
_SEQ = 16906


_BM = 512


def _proj_body(et_ref, w_ref, b_ref, out_ref):
    acc = lax.dot_general(
        et_ref[...], w_ref[...], (((0,), (0,)), ((), ())),
        preferred_element_type=jnp.float32,
    ) + b_ref[...]
    out_ref[...] = acc.reshape(_BM, 4, 128)


def _project(emb_t, w, b):
    return pl.pallas_call(
        _proj_body,
        grid=(pl.cdiv(_NUM_EMB, _BM),),
        in_specs=[
            pl.BlockSpec((_EMB_DIM, _BM), lambda i: (0, i)),
            pl.BlockSpec((_EMB_DIM, _OUT_DIM), lambda i: (0, 0)),
            pl.BlockSpec((1, _OUT_DIM), lambda i: (0, 0)),
        ],
        out_specs=pl.BlockSpec((_BM, 4, 128), lambda i: (i, 0, 0)),
        out_shape=jax.ShapeDtypeStruct((_NUM_EMB, 4, 128), jnp.float32),
    )(emb_t, w, b.reshape(1, _OUT_DIM))


_NW = 32
_TWIN = 24
_NWIN = 22
_CHUNK = _TWIN * _NWIN
_SEQ_PAD = 16912
_TAIL_T0 = _NW * _CHUNK
_TAIL_N = _SEQ - _TAIL_T0

_mesh = plsc.VectorSubcoreMesh(core_axis_name="c", subcore_axis_name="s")


@functools.partial(
    pl.kernel,
    out_type=jax.ShapeDtypeStruct((_SEQ, 4, _BATCH, 128), jnp.float32),
    mesh=_mesh,
    scratch_types=[
        pltpu.VMEM((_BATCH * (_CHUNK + 16),), jnp.int32),
        pltpu.VMEM((2, _TWIN, 4, _BATCH, 128), jnp.float32),
        pltpu.SemaphoreType.DMA,
        pltpu.SemaphoreType.DMA,
        pltpu.SemaphoreType.DMA,
        pltpu.SemaphoreType.DMA,
    ],
)
def _gather(table_hbm, idx_hbm, out_hbm, idxs, slab2, g0, g1, w0, w1):
    gsem = (g0, g1)
    wsem = (w0, w1)
    wid = lax.axis_index("s") * 2 + lax.axis_index("c")
    base = wid * _CHUNK
    is_last = wid == _NW - 1

    _IW = _CHUNK + 16
    for b in range(_BATCH):
        pltpu.async_copy(idx_hbm.at[pl.ds(b * _SEQ_PAD + base, _IW)],
                         idxs.at[pl.ds(b * _IW, _IW)], g0)
    for b in range(_BATCH):
        pltpu.make_async_copy(idx_hbm.at[pl.ds(b * _SEQ_PAD + base, _IW)],
                              idxs.at[pl.ds(b * _IW, _IW)], g0).wait()
    @pl.loop(0, _BATCH * _IW, step=16)
    def _norm(j):
        jj = pl.multiple_of(j, 16)
        v = idxs[pl.ds(jj, 16)]
        idxs[pl.ds(jj, 16)] = lax.rem(v + _NUM_EMB, _NUM_EMB)

    def fire_gathers(i, s):
        for b in range(_BATCH):
            pltpu.async_copy(
                table_hbm.at[idxs.at[pl.ds(b * (_CHUNK + 16) + i * _TWIN, _TWIN)]],
                slab2.at[s, :, :, b, :], gsem[s])

    def wait_gathers(s):
        for b in range(_BATCH):
            pltpu.make_async_copy(
                table_hbm.at[idxs.at[pl.ds(0, _TWIN)]],
                slab2.at[s, :, :, b, :], gsem[s]).wait()

    def fire_wb(i, s):
        pltpu.async_copy(slab2.at[s],
                         out_hbm.at[pl.ds(base + i * _TWIN, _TWIN)], wsem[s])

    def wait_wb(s):
        pltpu.make_async_copy(slab2.at[s], out_hbm.at[pl.ds(0, _TWIN)],
                              wsem[s]).wait()

    @pl.loop(0, _NWIN // 2)
    def _pair(k):
        a = 2 * k

        @pl.when(k > 0)
        def _():
            wait_wb(0)
        fire_gathers(a, 0)

        @pl.when(k > 0)
        def _():
            wait_gathers(1)
            fire_wb(a - 1, 1)
            wait_wb(1)
        fire_gathers(a + 1, 1)

        wait_gathers(0)
        fire_wb(a, 0)

    wait_gathers(1)
    fire_wb(_NWIN - 1, 1)
    wait_wb(0)
    wait_wb(1)

    @pl.when(is_last)
    def _tail():
        for b in range(_BATCH):
            pltpu.async_copy(table_hbm.at[idxs.at[pl.ds(b * (_CHUNK + 16) + _CHUNK, 16)]],
                             slab2.at[0, pl.ds(0, 16), :, b, :], g0)
        for b in range(_BATCH):
            pltpu.make_async_copy(table_hbm.at[idxs.at[pl.ds(b * (_CHUNK + 16) + _CHUNK, 16)]],
                                  slab2.at[0, pl.ds(0, 16), :, b, :],
                                  g0).wait()
        pltpu.sync_copy(slab2.at[0, pl.ds(0, _TAIL_N)],
                        out_hbm.at[pl.ds(_TAIL_T0, _TAIL_N)])


def kernel(x, emb, W, b):
    proj = _project(emb.T, W, b)
    idx = jnp.pad(x, ((0, 0), (0, _SEQ_PAD - _SEQ))).reshape(_BATCH * _SEQ_PAD)
    slab = _gather(proj, idx)
    return slab.transpose(2, 0, 1, 3).reshape(_BATCH, _SEQ, _OUT_DIM)


# speedup vs baseline: 2.4154x; 1.0001x over previous
import functools

import jax
import jax.numpy as jnp
from jax import lax
from jax.experimental import pallas as pl
from jax.experimental.pallas import tpu as pltpu
from jax.experimental.pallas import tpu_sc as plsc

_NUM_EMB = 16909
_EMB_DIM = 200
_OUT_DIM = 512
_BATCH = 4
_SEQ = 16906


_BM = 512


def _proj_body(et_ref, w_ref, b_ref, out_ref):
    acc = lax.dot_general(
        et_ref[...].astype(jnp.bfloat16), w_ref[...].astype(jnp.bfloat16),
        (((0,), (0,)), ((), ())),
        preferred_element_type=jnp.float32,
    ) + b_ref[...]
    out_ref[...] = acc.reshape(_BM, 4, 128)


def _project(emb_t, w, b):
    return pl.pallas_call(
        _proj_body,
        grid=(pl.cdiv(_NUM_EMB, _BM),),
        in_specs=[
            pl.BlockSpec((_EMB_DIM, _BM), lambda i: (0, i)),
            pl.BlockSpec((_EMB_DIM, _OUT_DIM), lambda i: (0, 0)),
            pl.BlockSpec((1, _OUT_DIM), lambda i: (0, 0)),
        ],
        out_specs=pl.BlockSpec((_BM, 4, 128), lambda i: (i, 0, 0)),
        out_shape=jax.ShapeDtypeStruct((_NUM_EMB, 4, 128), jnp.float32),
    )(emb_t, w, b.reshape(1, _OUT_DIM))


_NW = 32
_TWIN = 24
_NWIN = 22
_CHUNK = _TWIN * _NWIN
_SEQ_PAD = 16912
_TAIL_T0 = _NW * _CHUNK
_TAIL_N = _SEQ - _TAIL_T0

_mesh = plsc.VectorSubcoreMesh(core_axis_name="c", subcore_axis_name="s")


@functools.partial(
    pl.kernel,
    out_type=jax.ShapeDtypeStruct((_SEQ, 4, _BATCH, 128), jnp.float32),
    mesh=_mesh,
    scratch_types=[
        pltpu.VMEM((_BATCH * (_CHUNK + 16),), jnp.int32),
        pltpu.VMEM((2, _TWIN, 4, _BATCH, 128), jnp.float32),
        pltpu.SemaphoreType.DMA,
        pltpu.SemaphoreType.DMA,
        pltpu.SemaphoreType.DMA,
        pltpu.SemaphoreType.DMA,
    ],
)
def _gather(table_hbm, idx_hbm, out_hbm, idxs, slab2, g0, g1, w0, w1):
    gsem = (g0, g1)
    wsem = (w0, w1)
    wid = lax.axis_index("s") * 2 + lax.axis_index("c")
    base = wid * _CHUNK
    is_last = wid == _NW - 1

    _IW = _CHUNK + 16
    for b in range(_BATCH):
        pltpu.async_copy(idx_hbm.at[pl.ds(b * _SEQ_PAD + base, _IW)],
                         idxs.at[pl.ds(b * _IW, _IW)], g0)
    for b in range(_BATCH):
        pltpu.make_async_copy(idx_hbm.at[pl.ds(b * _SEQ_PAD + base, _IW)],
                              idxs.at[pl.ds(b * _IW, _IW)], g0).wait()
    @pl.loop(0, _BATCH * _IW, step=16)
    def _norm(j):
        jj = pl.multiple_of(j, 16)
        v = idxs[pl.ds(jj, 16)]
        idxs[pl.ds(jj, 16)] = lax.rem(v + _NUM_EMB, _NUM_EMB)

    def fire_gathers(i, s):
        for b in range(_BATCH):
            pltpu.async_copy(
                table_hbm.at[idxs.at[pl.ds(b * (_CHUNK + 16) + i * _TWIN, _TWIN)]],
                slab2.at[s, :, :, b, :], gsem[s])

    def wait_gathers(s):
        for b in range(_BATCH):
            pltpu.make_async_copy(
                table_hbm.at[idxs.at[pl.ds(0, _TWIN)]],
                slab2.at[s, :, :, b, :], gsem[s]).wait()

    def fire_wb(i, s):
        pltpu.async_copy(slab2.at[s],
                         out_hbm.at[pl.ds(base + i * _TWIN, _TWIN)], wsem[s])

    def wait_wb(s):
        pltpu.make_async_copy(slab2.at[s], out_hbm.at[pl.ds(0, _TWIN)],
                              wsem[s]).wait()

    @pl.loop(0, _NWIN // 2)
    def _pair(k):
        a = 2 * k

        @pl.when(k > 0)
        def _():
            wait_wb(0)
        fire_gathers(a, 0)

        @pl.when(k > 0)
        def _():
            wait_gathers(1)
            fire_wb(a - 1, 1)
            wait_wb(1)
        fire_gathers(a + 1, 1)

        wait_gathers(0)
        fire_wb(a, 0)

    wait_gathers(1)
    fire_wb(_NWIN - 1, 1)
    wait_wb(0)
    wait_wb(1)

    @pl.when(is_last)
    def _tail():
        for b in range(_BATCH):
            pltpu.async_copy(table_hbm.at[idxs.at[pl.ds(b * (_CHUNK + 16) + _CHUNK, 16)]],
                             slab2.at[0, pl.ds(0, 16), :, b, :], g0)
        for b in range(_BATCH):
            pltpu.make_async_copy(table_hbm.at[idxs.at[pl.ds(b * (_CHUNK + 16) + _CHUNK, 16)]],
                                  slab2.at[0, pl.ds(0, 16), :, b, :],
                                  g0).wait()
        pltpu.sync_copy(slab2.at[0, pl.ds(0, _TAIL_N)],
                        out_hbm.at[pl.ds(_TAIL_T0, _TAIL_N)])


def kernel(x, emb, W, b):
    proj = _project(emb.T, W, b)
    idx = jnp.pad(x, ((0, 0), (0, _SEQ_PAD - _SEQ))).reshape(_BATCH * _SEQ_PAD)
    slab = _gather(proj, idx)
    return slab.transpose(2, 0, 1, 3).reshape(_BATCH, _SEQ, _OUT_DIM)


# speedup vs baseline: 2.6539x; 1.0988x over previous
import functools

import jax
import jax.numpy as jnp
from jax import lax
from jax.experimental import pallas as pl
from jax.experimental.pallas import tpu as pltpu
from jax.experimental.pallas import tpu_sc as plsc

_NUM_EMB = 16909
_EMB_DIM = 200
_OUT_DIM = 512
_BATCH = 4
_SEQ = 16906


_BM = 2048


def _proj_body(et_ref, w_ref, b_ref, out_ref):
    acc = lax.dot_general(
        et_ref[...], w_ref[...], (((0,), (0,)), ((), ())),
        preferred_element_type=jnp.float32,
    ) + b_ref[...]
    out_ref[...] = acc.reshape(_BM, 4, 128)


def _project(emb_t, w, b):
    return pl.pallas_call(
        _proj_body,
        grid=(pl.cdiv(_NUM_EMB, _BM),),
        in_specs=[
            pl.BlockSpec((_EMB_DIM, _BM), lambda i: (0, i)),
            pl.BlockSpec((_EMB_DIM, _OUT_DIM), lambda i: (0, 0)),
            pl.BlockSpec((1, _OUT_DIM), lambda i: (0, 0)),
        ],
        out_specs=pl.BlockSpec((_BM, 4, 128), lambda i: (i, 0, 0)),
        out_shape=jax.ShapeDtypeStruct((_NUM_EMB, 4, 128), jnp.float32),
    )(emb_t, w, b.reshape(1, _OUT_DIM))


_NW = 32
_TWIN = 24
_NWIN = 22
_CHUNK = _TWIN * _NWIN
_SEQ_PAD = 16912
_TAIL_T0 = _NW * _CHUNK
_TAIL_N = _SEQ - _TAIL_T0

_mesh = plsc.VectorSubcoreMesh(core_axis_name="c", subcore_axis_name="s")


@functools.partial(
    pl.kernel,
    out_type=jax.ShapeDtypeStruct((_SEQ, 4, _BATCH, 128), jnp.float32),
    mesh=_mesh,
    scratch_types=[
        pltpu.VMEM((_BATCH * (_CHUNK + 16),), jnp.int32),
        pltpu.VMEM((2, _TWIN, 4, _BATCH, 128), jnp.float32),
        pltpu.SemaphoreType.DMA,
        pltpu.SemaphoreType.DMA,
        pltpu.SemaphoreType.DMA,
        pltpu.SemaphoreType.DMA,
    ],
)
def _gather(table_hbm, idx_hbm, out_hbm, idxs, slab2, g0, g1, w0, w1):
    gsem = (g0, g1)
    wsem = (w0, w1)
    wid = lax.axis_index("s") * 2 + lax.axis_index("c")
    base = wid * _CHUNK
    is_last = wid == _NW - 1

    _IW = _CHUNK + 16
    for b in range(_BATCH):
        pltpu.async_copy(idx_hbm.at[pl.ds(b * _SEQ_PAD + base, _IW)],
                         idxs.at[pl.ds(b * _IW, _IW)], g0)
    for b in range(_BATCH):
        pltpu.make_async_copy(idx_hbm.at[pl.ds(b * _SEQ_PAD + base, _IW)],
                              idxs.at[pl.ds(b * _IW, _IW)], g0).wait()
    @pl.loop(0, _BATCH * _IW, step=16)
    def _norm(j):
        jj = pl.multiple_of(j, 16)
        v = idxs[pl.ds(jj, 16)]
        idxs[pl.ds(jj, 16)] = lax.rem(v + _NUM_EMB, _NUM_EMB)

    def fire_gathers(i, s):
        for b in range(_BATCH):
            pltpu.async_copy(
                table_hbm.at[idxs.at[pl.ds(b * (_CHUNK + 16) + i * _TWIN, _TWIN)]],
                slab2.at[s, :, :, b, :], gsem[s])

    def wait_gathers(s):
        for b in range(_BATCH):
            pltpu.make_async_copy(
                table_hbm.at[idxs.at[pl.ds(0, _TWIN)]],
                slab2.at[s, :, :, b, :], gsem[s]).wait()

    def fire_wb(i, s):
        pltpu.async_copy(slab2.at[s],
                         out_hbm.at[pl.ds(base + i * _TWIN, _TWIN)], wsem[s])

    def wait_wb(s):
        pltpu.make_async_copy(slab2.at[s], out_hbm.at[pl.ds(0, _TWIN)],
                              wsem[s]).wait()

    @pl.loop(0, _NWIN // 2)
    def _pair(k):
        a = 2 * k

        @pl.when(k > 0)
        def _():
            wait_wb(0)
        fire_gathers(a, 0)

        @pl.when(k > 0)
        def _():
            wait_gathers(1)
            fire_wb(a - 1, 1)
            wait_wb(1)
        fire_gathers(a + 1, 1)

        wait_gathers(0)
        fire_wb(a, 0)

    wait_gathers(1)
    fire_wb(_NWIN - 1, 1)
    wait_wb(0)
    wait_wb(1)

    @pl.when(is_last)
    def _tail():
        for b in range(_BATCH):
            pltpu.async_copy(table_hbm.at[idxs.at[pl.ds(b * (_CHUNK + 16) + _CHUNK, 16)]],
                             slab2.at[0, pl.ds(0, 16), :, b, :], g0)
        for b in range(_BATCH):
            pltpu.make_async_copy(table_hbm.at[idxs.at[pl.ds(b * (_CHUNK + 16) + _CHUNK, 16)]],
                                  slab2.at[0, pl.ds(0, 16), :, b, :],
                                  g0).wait()
        pltpu.sync_copy(slab2.at[0, pl.ds(0, _TAIL_N)],
                        out_hbm.at[pl.ds(_TAIL_T0, _TAIL_N)])


def kernel(x, emb, W, b):
    proj = _project(emb.T, W, b)
    idx = jnp.pad(x, ((0, 0), (0, _SEQ_PAD - _SEQ))).reshape(_BATCH * _SEQ_PAD)
    slab = _gather(proj, idx)
    return slab.transpose(2, 0, 1, 3).reshape(_BATCH, _SEQ, _OUT_DIM)


# speedup vs baseline: 2.6828x; 1.0109x over previous
import functools

import jax
import jax.numpy as jnp
from jax import lax
from jax.experimental import pallas as pl
from jax.experimental.pallas import tpu as pltpu
from jax.experimental.pallas import tpu_sc as plsc

_NUM_EMB = 16909
_EMB_DIM = 200
_OUT_DIM = 512
_BATCH = 4
_SEQ = 16906


_BM = 4096


def _proj_body(et_ref, w_ref, b_ref, out_ref):
    acc = lax.dot_general(
        et_ref[...], w_ref[...], (((0,), (0,)), ((), ())),
        preferred_element_type=jnp.float32,
    ) + b_ref[...]
    out_ref[...] = acc.reshape(_BM, 4, 128)


def _project(emb_t, w, b):
    return pl.pallas_call(
        _proj_body,
        grid=(pl.cdiv(_NUM_EMB, _BM),),
        in_specs=[
            pl.BlockSpec((_EMB_DIM, _BM), lambda i: (0, i)),
            pl.BlockSpec((_EMB_DIM, _OUT_DIM), lambda i: (0, 0)),
            pl.BlockSpec((1, _OUT_DIM), lambda i: (0, 0)),
        ],
        out_specs=pl.BlockSpec((_BM, 4, 128), lambda i: (i, 0, 0)),
        out_shape=jax.ShapeDtypeStruct((_NUM_EMB, 4, 128), jnp.float32),
    )(emb_t, w, b.reshape(1, _OUT_DIM))


_NW = 32
_TWIN = 24
_NWIN = 22
_CHUNK = _TWIN * _NWIN
_SEQ_PAD = 16912
_TAIL_T0 = _NW * _CHUNK
_TAIL_N = _SEQ - _TAIL_T0

_mesh = plsc.VectorSubcoreMesh(core_axis_name="c", subcore_axis_name="s")


@functools.partial(
    pl.kernel,
    out_type=jax.ShapeDtypeStruct((_SEQ, 4, _BATCH, 128), jnp.float32),
    mesh=_mesh,
    scratch_types=[
        pltpu.VMEM((_BATCH * (_CHUNK + 16),), jnp.int32),
        pltpu.VMEM((2, _TWIN, 4, _BATCH, 128), jnp.float32),
        pltpu.SemaphoreType.DMA,
        pltpu.SemaphoreType.DMA,
        pltpu.SemaphoreType.DMA,
        pltpu.SemaphoreType.DMA,
    ],
)
def _gather(table_hbm, idx_hbm, out_hbm, idxs, slab2, g0, g1, w0, w1):
    gsem = (g0, g1)
    wsem = (w0, w1)
    wid = lax.axis_index("s") * 2 + lax.axis_index("c")
    base = wid * _CHUNK
    is_last = wid == _NW - 1

    _IW = _CHUNK + 16
    for b in range(_BATCH):
        pltpu.async_copy(idx_hbm.at[pl.ds(b * _SEQ_PAD + base, _IW)],
                         idxs.at[pl.ds(b * _IW, _IW)], g0)
    for b in range(_BATCH):
        pltpu.make_async_copy(idx_hbm.at[pl.ds(b * _SEQ_PAD + base, _IW)],
                              idxs.at[pl.ds(b * _IW, _IW)], g0).wait()
    @pl.loop(0, _BATCH * _IW, step=16)
    def _norm(j):
        jj = pl.multiple_of(j, 16)
        v = idxs[pl.ds(jj, 16)]
        idxs[pl.ds(jj, 16)] = lax.rem(v + _NUM_EMB, _NUM_EMB)

    def fire_gathers(i, s):
        for b in range(_BATCH):
            pltpu.async_copy(
                table_hbm.at[idxs.at[pl.ds(b * (_CHUNK + 16) + i * _TWIN, _TWIN)]],
                slab2.at[s, :, :, b, :], gsem[s])

    def wait_gathers(s):
        for b in range(_BATCH):
            pltpu.make_async_copy(
                table_hbm.at[idxs.at[pl.ds(0, _TWIN)]],
                slab2.at[s, :, :, b, :], gsem[s]).wait()

    def fire_wb(i, s):
        pltpu.async_copy(slab2.at[s],
                         out_hbm.at[pl.ds(base + i * _TWIN, _TWIN)], wsem[s])

    def wait_wb(s):
        pltpu.make_async_copy(slab2.at[s], out_hbm.at[pl.ds(0, _TWIN)],
                              wsem[s]).wait()

    @pl.loop(0, _NWIN // 2)
    def _pair(k):
        a = 2 * k

        @pl.when(k > 0)
        def _():
            wait_wb(0)
        fire_gathers(a, 0)

        @pl.when(k > 0)
        def _():
            wait_gathers(1)
            fire_wb(a - 1, 1)
            wait_wb(1)
        fire_gathers(a + 1, 1)

        wait_gathers(0)
        fire_wb(a, 0)

    wait_gathers(1)
    fire_wb(_NWIN - 1, 1)
    wait_wb(0)
    wait_wb(1)

    @pl.when(is_last)
    def _tail():
        for b in range(_BATCH):
            pltpu.async_copy(table_hbm.at[idxs.at[pl.ds(b * (_CHUNK + 16) + _CHUNK, 16)]],
                             slab2.at[0, pl.ds(0, 16), :, b, :], g0)
        for b in range(_BATCH):
            pltpu.make_async_copy(table_hbm.at[idxs.at[pl.ds(b * (_CHUNK + 16) + _CHUNK, 16)]],
                                  slab2.at[0, pl.ds(0, 16), :, b, :],
                                  g0).wait()
        pltpu.sync_copy(slab2.at[0, pl.ds(0, _TAIL_N)],
                        out_hbm.at[pl.ds(_TAIL_T0, _TAIL_N)])


def kernel(x, emb, W, b):
    proj = _project(emb.T, W, b)
    idx = jnp.pad(x, ((0, 0), (0, _SEQ_PAD - _SEQ))).reshape(_BATCH * _SEQ_PAD)
    slab = _gather(proj, idx)
    return slab.transpose(2, 0, 1, 3).reshape(_BATCH, _SEQ, _OUT_DIM)
